# Initial kernel scaffold; baseline (speedup 1.0000x reference)
import jax
import jax.numpy as jnp
from jax.experimental import pallas as pl


def kernel(edge_src, edge_dst, edge_type, embed, h_bias1, W2, b2, W3, b3):
    raise NotImplementedError("write your pallas kernel here")


# speedup vs baseline: 5.7269x; 5.7269x over previous
import functools

import jax
import jax.numpy as jnp
from jax import lax
from jax.experimental import pallas as pl
from jax.experimental.pallas import tpu as pltpu, tpu_sc as plsc

N = 10000
E = 320000
R = 4
RN = R * N
D = 128
NC = 2
NS = 16
LW = 128
EPAD = 327680
EROWS = EPAD // LW
ROWS_PER_SUB = EROWS // NS
ROWS_PER_WRK = EROWS // (NC * NS)
RNP = 40960
NSLICE = RNP // NS

_mesh = plsc.VectorSubcoreMesh(core_axis_name="c", subcore_axis_name="s")


@functools.partial(
    pl.kernel,
    out_type=jax.ShapeDtypeStruct((NC, RNP, 8), jnp.float32),
    mesh=_mesh,
    scratch_types=[
        pltpu.VMEM((ROWS_PER_WRK, LW), jnp.int32),
        pltpu.VMEM((LW, 8), jnp.float32),
        pltpu.VMEM_SHARED((RNP, 8), jnp.float32),
    ],
    compiler_params=pltpu.CompilerParams(use_tc_tiling_on_sc=False),
)
def _deg_kernel(dsti_hbm, ones_hbm, zeros_hbm, out_hbm, dsti_v, ones_v, acc_sh):
    c = lax.axis_index("c")
    s = lax.axis_index("s")
    wid = s * NC + c
    pltpu.sync_copy(zeros_hbm.at[pl.ds(s * NSLICE, NSLICE)],
                    acc_sh.at[pl.ds(s * NSLICE, NSLICE)])
    pltpu.sync_copy(dsti_hbm.at[pl.ds(wid * ROWS_PER_WRK, ROWS_PER_WRK)], dsti_v)
    pltpu.sync_copy(ones_hbm, ones_v)
    plsc.subcore_barrier()

    def body(j, carry):
        pltpu.sync_copy(ones_v, acc_sh.at[dsti_v.at[j]], add=True)
        return carry

    lax.fori_loop(0, ROWS_PER_WRK, body, 0, unroll=False)
    plsc.subcore_barrier()
    pltpu.sync_copy(acc_sh.at[pl.ds(s * NSLICE, NSLICE)],
                    out_hbm.at[c, pl.ds(s * NSLICE, NSLICE)])


@functools.partial(
    pl.kernel,
    out_type=jax.ShapeDtypeStruct((NC, RNP, 32), jnp.float32),
    mesh=_mesh,
    scratch_types=[
        pltpu.VMEM((ROWS_PER_SUB, LW), jnp.int32),
        pltpu.VMEM((ROWS_PER_SUB, LW), jnp.int32),
        pltpu.VMEM((LW, 32), jnp.float32),
        pltpu.VMEM_SHARED((RNP, 32), jnp.float32),
    ],
    compiler_params=pltpu.CompilerParams(use_tc_tiling_on_sc=False),
)
def _scatter_kernel(table_hbm, srci_hbm, dsti_hbm, zeros_hbm, out_hbm,
                    srci_v, dsti_v, rows_v, acc_sh):
    c = lax.axis_index("c")
    s = lax.axis_index("s")
    pltpu.sync_copy(zeros_hbm.at[pl.ds(s * NSLICE, NSLICE)],
                    acc_sh.at[pl.ds(s * NSLICE, NSLICE)])
    pltpu.sync_copy(srci_hbm.at[c, pl.ds(s * ROWS_PER_SUB, ROWS_PER_SUB)], srci_v)
    pltpu.sync_copy(dsti_hbm.at[pl.ds(s * ROWS_PER_SUB, ROWS_PER_SUB)], dsti_v)
    plsc.subcore_barrier()

    def body(j, carry):
        pltpu.sync_copy(table_hbm.at[srci_v.at[j]], rows_v)
        pltpu.sync_copy(rows_v, acc_sh.at[dsti_v.at[j]], add=True)
        return carry

    lax.fori_loop(0, ROWS_PER_SUB, body, 0, unroll=False)
    plsc.subcore_barrier()
    pltpu.sync_copy(acc_sh.at[pl.ds(s * NSLICE, NSLICE)],
                    out_hbm.at[c, pl.ds(s * NSLICE, NSLICE)])


_NB = 1000


def _l0_body(acc_ref, degp_ref, b_ref, h_ref, inv_ref):
    deg = degp_ref[0] + degp_ref[1]
    inv = 1.0 / jnp.maximum(deg, 1.0)
    h = jnp.zeros((_NB, D), jnp.float32)
    for r in range(R):
        h = h + acc_ref[r] * inv[r]
    h_ref[...] = jnp.maximum(h + b_ref[...], 0.0)
    inv_ref[...] = inv


def _l0_dense(acc, degp, b1):
    return pl.pallas_call(
        _l0_body,
        grid=(N // _NB,),
        in_specs=[
            pl.BlockSpec((R, _NB, D), lambda i: (0, i, 0)),
            pl.BlockSpec((NC, R, _NB, 1), lambda i: (0, 0, i, 0)),
            pl.BlockSpec((1, D), lambda i: (0, 0)),
        ],
        out_specs=[
            pl.BlockSpec((_NB, D), lambda i: (i, 0)),
            pl.BlockSpec((R, _NB, 1), lambda i: (0, i, 0)),
        ],
        out_shape=[
            jax.ShapeDtypeStruct((N, D), jnp.float32),
            jax.ShapeDtypeStruct((R, N, 1), jnp.float32),
        ],
    )(acc, degp, b1)


def _mm_body(relu, k, acc_ref, inv_ref, w_ref, b_ref, o_ref):
    y = jnp.zeros((_NB, k), jnp.float32)
    for r in range(R):
        x = acc_ref[r] * inv_ref[r]
        y = y + jnp.dot(x, w_ref[r], preferred_element_type=jnp.float32)
    y = y + b_ref[...]
    o_ref[...] = jnp.maximum(y, 0.0) if relu else y


def _mm_dense(acc, inv, w, b, relu):
    k = w.shape[-1]
    return pl.pallas_call(
        functools.partial(_mm_body, relu, k),
        grid=(N // _NB,),
        in_specs=[
            pl.BlockSpec((R, _NB, D), lambda i: (0, i, 0)),
            pl.BlockSpec((R, _NB, 1), lambda i: (0, i, 0)),
            pl.BlockSpec((R, D, k), lambda i: (0, 0, 0)),
            pl.BlockSpec((1, k), lambda i: (0, 0)),
        ],
        out_specs=pl.BlockSpec((_NB, k), lambda i: (i, 0)),
        out_shape=jax.ShapeDtypeStruct((N, k), jnp.float32),
    )(acc, inv, w, b)


def _aggregate(x, srci, dsti, zeros32):
    chunks = []
    for p in range(2):
        table = jnp.concatenate(
            [x[:, 64 * p:64 * p + 32], x[:, 64 * p + 32:64 * p + 64]], axis=0)
        chunks.append(_scatter_kernel(table, srci, dsti, zeros32))
    acc = jnp.concatenate(
        [chunks[0][0, :RN], chunks[0][1, :RN],
         chunks[1][0, :RN], chunks[1][1, :RN]], axis=-1)
    return acc.reshape(R, N, D)


def kernel(edge_src, edge_dst, edge_type, embed, h_bias1, W2, b2, W3, b3):
    pad = EPAD - E
    src_p = jnp.concatenate([edge_src, jnp.zeros((pad,), jnp.int32)])
    srci = jnp.stack([src_p, src_p + N]).reshape(NC, EROWS, LW)
    dst_p = jnp.concatenate(
        [edge_type * N + edge_dst, jnp.full((pad,), RN, jnp.int32)])
    dsti = dst_p.reshape(EROWS, LW)

    zeros32 = jnp.zeros((RNP, 32), jnp.float32)
    zeros8 = jnp.zeros((RNP, 8), jnp.float32)
    ones8 = jnp.ones((LW, 8), jnp.float32)

    degp = _deg_kernel(dsti, ones8, zeros8)
    degp = degp[:, :RN, 0].reshape(NC, R, N, 1)

    acc0 = _aggregate(embed, srci, dsti, zeros32)
    h0, inv = _l0_dense(acc0, degp, h_bias1.reshape(1, D))

    acc1 = _aggregate(h0, srci, dsti, zeros32)
    h1 = _mm_dense(acc1, inv, W2, b2.reshape(1, D), relu=True)

    acc2 = _aggregate(h1, srci, dsti, zeros32)
    out = _mm_dense(acc2, inv, W3, b3.reshape(1, 16), relu=False)
    return out


# speedup vs baseline: 6.8932x; 1.2036x over previous
import functools

import jax
import jax.numpy as jnp
from jax import lax
from jax.experimental import pallas as pl
from jax.experimental.pallas import tpu as pltpu, tpu_sc as plsc

N = 10000
E = 320000
R = 4
RN = R * N
D = 128
NC = 2
NS = 16
LW = 128
EPAD = 327680
EROWS = EPAD // LW
ROWS_PER_SUB = EROWS // NS
ROWS_PER_WRK = EROWS // (NC * NS)
RNP = 40960
NSLICE = RNP // NS

_mesh = plsc.VectorSubcoreMesh(core_axis_name="c", subcore_axis_name="s")


@functools.partial(
    pl.kernel,
    out_type=jax.ShapeDtypeStruct((NC, RNP, 8), jnp.float32),
    mesh=_mesh,
    scratch_types=[
        pltpu.VMEM((ROWS_PER_WRK, LW), jnp.int32),
        pltpu.VMEM((LW, 8), jnp.float32),
        pltpu.VMEM_SHARED((RNP, 8), jnp.float32),
    ],
    compiler_params=pltpu.CompilerParams(use_tc_tiling_on_sc=False),
)
def _deg_kernel(dsti_hbm, ones_hbm, zeros_hbm, out_hbm, dsti_v, ones_v, acc_sh):
    c = lax.axis_index("c")
    s = lax.axis_index("s")
    wid = s * NC + c
    pltpu.sync_copy(zeros_hbm.at[pl.ds(s * NSLICE, NSLICE)],
                    acc_sh.at[pl.ds(s * NSLICE, NSLICE)])
    pltpu.sync_copy(dsti_hbm.at[pl.ds(wid * ROWS_PER_WRK, ROWS_PER_WRK)], dsti_v)
    pltpu.sync_copy(ones_hbm, ones_v)
    plsc.subcore_barrier()

    def body(j, carry):
        pltpu.sync_copy(ones_v, acc_sh.at[dsti_v.at[j]], add=True)
        return carry

    lax.fori_loop(0, ROWS_PER_WRK, body, 0, unroll=False)
    plsc.subcore_barrier()
    pltpu.sync_copy(acc_sh.at[pl.ds(s * NSLICE, NSLICE)],
                    out_hbm.at[c, pl.ds(s * NSLICE, NSLICE)])


@functools.partial(
    pl.kernel,
    out_type=jax.ShapeDtypeStruct((NC, RNP, 32), jnp.float32),
    mesh=_mesh,
    scratch_types=[
        pltpu.VMEM((ROWS_PER_SUB, LW), jnp.int32),
        pltpu.VMEM((ROWS_PER_SUB, LW), jnp.int32),
        pltpu.VMEM((LW, 32), jnp.float32),
        pltpu.VMEM((LW, 32), jnp.float32),
        pltpu.VMEM_SHARED((RNP, 32), jnp.float32),
        pltpu.SemaphoreType.DMA,
        pltpu.SemaphoreType.DMA,
        pltpu.SemaphoreType.DMA,
        pltpu.SemaphoreType.DMA,
    ],
    compiler_params=pltpu.CompilerParams(use_tc_tiling_on_sc=False),
)
def _scatter_kernel(table_hbm, srci_hbm, dsti_hbm, zeros_hbm, out_hbm,
                    srci_v, dsti_v, rows0_v, rows1_v, acc_sh,
                    g0, g1, s0, s1):
    c = lax.axis_index("c")
    s = lax.axis_index("s")
    pltpu.sync_copy(zeros_hbm.at[pl.ds(s * NSLICE, NSLICE)],
                    acc_sh.at[pl.ds(s * NSLICE, NSLICE)])
    pltpu.sync_copy(srci_hbm.at[c, pl.ds(s * ROWS_PER_SUB, ROWS_PER_SUB)], srci_v)
    pltpu.sync_copy(dsti_hbm.at[pl.ds(s * ROWS_PER_SUB, ROWS_PER_SUB)], dsti_v)
    plsc.subcore_barrier()

    def gather(j, buf, sem):
        pltpu.async_copy(table_hbm.at[srci_v.at[j]], buf, sem)

    def gwait(j, buf, sem):
        pltpu.make_async_copy(table_hbm.at[srci_v.at[j]], buf, sem).wait()

    def scat(j, buf, sem):
        pltpu.async_copy(buf, acc_sh.at[dsti_v.at[j]], sem, add=True)

    def swait(j, buf, sem):
        pltpu.make_async_copy(buf, acc_sh.at[dsti_v.at[j]], sem).wait()

    gather(0, rows0_v, g0)
    gather(1, rows1_v, g1)

    def body(k, carry):
        j0 = 2 * k
        j1 = j0 + 1
        gwait(j0, rows0_v, g0)
        scat(j0, rows0_v, s0)
        gwait(j1, rows1_v, g1)
        scat(j1, rows1_v, s1)
        swait(j0, rows0_v, s0)

        @pl.when(k < ROWS_PER_SUB // 2 - 1)
        def _():
            gather(j0 + 2, rows0_v, g0)

        swait(j1, rows1_v, s1)

        @pl.when(k < ROWS_PER_SUB // 2 - 1)
        def _():
            gather(j1 + 2, rows1_v, g1)

        return carry

    lax.fori_loop(0, ROWS_PER_SUB // 2, body, 0, unroll=False)
    plsc.subcore_barrier()
    pltpu.sync_copy(acc_sh.at[pl.ds(s * NSLICE, NSLICE)],
                    out_hbm.at[c, pl.ds(s * NSLICE, NSLICE)])


_NB = 1000


def _l0_body(acc_ref, degp_ref, b_ref, h_ref, inv_ref):
    deg = degp_ref[0] + degp_ref[1]
    inv = 1.0 / jnp.maximum(deg, 1.0)
    h = jnp.zeros((_NB, D), jnp.float32)
    for r in range(R):
        h = h + acc_ref[r] * inv[r]
    h_ref[...] = jnp.maximum(h + b_ref[...], 0.0)
    inv_ref[...] = inv


def _l0_dense(acc, degp, b1):
    return pl.pallas_call(
        _l0_body,
        grid=(N // _NB,),
        in_specs=[
            pl.BlockSpec((R, _NB, D), lambda i: (0, i, 0)),
            pl.BlockSpec((NC, R, _NB, 1), lambda i: (0, 0, i, 0)),
            pl.BlockSpec((1, D), lambda i: (0, 0)),
        ],
        out_specs=[
            pl.BlockSpec((_NB, D), lambda i: (i, 0)),
            pl.BlockSpec((R, _NB, 1), lambda i: (0, i, 0)),
        ],
        out_shape=[
            jax.ShapeDtypeStruct((N, D), jnp.float32),
            jax.ShapeDtypeStruct((R, N, 1), jnp.float32),
        ],
    )(acc, degp, b1)


def _mm_body(relu, k, acc_ref, inv_ref, w_ref, b_ref, o_ref):
    y = jnp.zeros((_NB, k), jnp.float32)
    for r in range(R):
        x = acc_ref[r] * inv_ref[r]
        y = y + jnp.dot(x, w_ref[r], preferred_element_type=jnp.float32)
    y = y + b_ref[...]
    o_ref[...] = jnp.maximum(y, 0.0) if relu else y


def _mm_dense(acc, inv, w, b, relu):
    k = w.shape[-1]
    return pl.pallas_call(
        functools.partial(_mm_body, relu, k),
        grid=(N // _NB,),
        in_specs=[
            pl.BlockSpec((R, _NB, D), lambda i: (0, i, 0)),
            pl.BlockSpec((R, _NB, 1), lambda i: (0, i, 0)),
            pl.BlockSpec((R, D, k), lambda i: (0, 0, 0)),
            pl.BlockSpec((1, k), lambda i: (0, 0)),
        ],
        out_specs=pl.BlockSpec((_NB, k), lambda i: (i, 0)),
        out_shape=jax.ShapeDtypeStruct((N, k), jnp.float32),
    )(acc, inv, w, b)


def _aggregate(x, srci, dsti, zeros32):
    chunks = []
    for p in range(2):
        table = jnp.concatenate(
            [x[:, 64 * p:64 * p + 32], x[:, 64 * p + 32:64 * p + 64]], axis=0)
        chunks.append(_scatter_kernel(table, srci, dsti, zeros32))
    acc = jnp.concatenate(
        [chunks[0][0, :RN], chunks[0][1, :RN],
         chunks[1][0, :RN], chunks[1][1, :RN]], axis=-1)
    return acc.reshape(R, N, D)


def kernel(edge_src, edge_dst, edge_type, embed, h_bias1, W2, b2, W3, b3):
    pad = EPAD - E
    src_p = jnp.concatenate([edge_src, jnp.zeros((pad,), jnp.int32)])
    srci = jnp.stack([src_p, src_p + N]).reshape(NC, EROWS, LW)
    dst_p = jnp.concatenate(
        [edge_type * N + edge_dst, jnp.full((pad,), RN, jnp.int32)])
    dsti = dst_p.reshape(EROWS, LW)

    zeros32 = jnp.zeros((RNP, 32), jnp.float32)
    zeros8 = jnp.zeros((RNP, 8), jnp.float32)
    ones8 = jnp.ones((LW, 8), jnp.float32)

    degp = _deg_kernel(dsti, ones8, zeros8)
    degp = degp[:, :RN, 0].reshape(NC, R, N, 1)

    acc0 = _aggregate(embed, srci, dsti, zeros32)
    h0, inv = _l0_dense(acc0, degp, h_bias1.reshape(1, D))

    acc1 = _aggregate(h0, srci, dsti, zeros32)
    h1 = _mm_dense(acc1, inv, W2, b2.reshape(1, D), relu=True)

    acc2 = _aggregate(h1, srci, dsti, zeros32)
    out = _mm_dense(acc2, inv, W3, b3.reshape(1, 16), relu=False)
    return out


# speedup vs baseline: 9.0958x; 1.3195x over previous
import functools

import jax
import jax.numpy as jnp
from jax import lax
from jax.experimental import pallas as pl
from jax.experimental.pallas import tpu as pltpu, tpu_sc as plsc

N = 10000
E = 320000
R = 4
RN = R * N
D = 128
NC = 2
NS = 16
LW = 128
EPAD = 327680
EROWS = EPAD // LW
ROWS_PER_SUB = EROWS // NS
ROWS_PER_WRK = EROWS // (NC * NS)
RNP = 40960
NSLICE = RNP // NS

_mesh = plsc.VectorSubcoreMesh(core_axis_name="c", subcore_axis_name="s")


@functools.partial(
    pl.kernel,
    out_type=jax.ShapeDtypeStruct((NC, RNP, 8), jnp.float32),
    mesh=_mesh,
    scratch_types=[
        pltpu.VMEM((ROWS_PER_WRK, LW), jnp.int32),
        pltpu.VMEM((LW, 8), jnp.float32),
        pltpu.VMEM_SHARED((RNP, 8), jnp.float32),
    ],
    compiler_params=pltpu.CompilerParams(use_tc_tiling_on_sc=False),
)
def _deg_kernel(dsti_hbm, ones_hbm, zeros_hbm, out_hbm, dsti_v, ones_v, acc_sh):
    c = lax.axis_index("c")
    s = lax.axis_index("s")
    wid = s * NC + c
    pltpu.sync_copy(zeros_hbm.at[pl.ds(s * NSLICE, NSLICE)],
                    acc_sh.at[pl.ds(s * NSLICE, NSLICE)])
    pltpu.sync_copy(dsti_hbm.at[pl.ds(wid * ROWS_PER_WRK, ROWS_PER_WRK)], dsti_v)
    pltpu.sync_copy(ones_hbm, ones_v)
    plsc.subcore_barrier()

    def body(j, carry):
        pltpu.sync_copy(ones_v, acc_sh.at[dsti_v.at[j]], add=True)
        return carry

    lax.fori_loop(0, ROWS_PER_WRK, body, 0, unroll=False)
    plsc.subcore_barrier()
    pltpu.sync_copy(acc_sh.at[pl.ds(s * NSLICE, NSLICE)],
                    out_hbm.at[c, pl.ds(s * NSLICE, NSLICE)])


@functools.partial(
    pl.kernel,
    out_type=jax.ShapeDtypeStruct((NC, RNP, 32), jnp.float32),
    mesh=_mesh,
    scratch_types=[
        pltpu.VMEM((ROWS_PER_SUB, LW), jnp.int32),
        pltpu.VMEM((ROWS_PER_SUB, LW), jnp.int32),
        pltpu.VMEM((LW, 32), jnp.float32),
        pltpu.VMEM((LW, 32), jnp.float32),
        pltpu.VMEM_SHARED((RNP, 32), jnp.float32),
        pltpu.SemaphoreType.DMA,
        pltpu.SemaphoreType.DMA,
        pltpu.SemaphoreType.DMA,
        pltpu.SemaphoreType.DMA,
    ],
    compiler_params=pltpu.CompilerParams(use_tc_tiling_on_sc=False),
)
def _scatter_kernel(table_hbm, srci_hbm, dsti_hbm, zeros_hbm, out_hbm,
                    srci_v, dsti_v, rows0_v, rows1_v, acc_sh,
                    g0, g1, s0, s1):
    c = lax.axis_index("c")
    s = lax.axis_index("s")
    pltpu.sync_copy(zeros_hbm.at[pl.ds(s * NSLICE, NSLICE)],
                    acc_sh.at[pl.ds(s * NSLICE, NSLICE)])
    pltpu.sync_copy(srci_hbm.at[c, pl.ds(s * ROWS_PER_SUB, ROWS_PER_SUB)], srci_v)
    pltpu.sync_copy(dsti_hbm.at[pl.ds(s * ROWS_PER_SUB, ROWS_PER_SUB)], dsti_v)
    plsc.subcore_barrier()

    def gather(j, buf, sem):
        pltpu.async_copy(table_hbm.at[srci_v.at[j]], buf, sem)

    def gwait(j, buf, sem):
        pltpu.make_async_copy(table_hbm.at[srci_v.at[j]], buf, sem).wait()

    def scat(j, buf, sem):
        pltpu.async_copy(buf, acc_sh.at[dsti_v.at[j]], sem, add=True)

    def swait(j, buf, sem):
        pltpu.make_async_copy(buf, acc_sh.at[dsti_v.at[j]], sem).wait()

    gather(0, rows0_v, g0)
    gather(1, rows1_v, g1)

    def body(k, carry):
        j0 = 2 * k
        j1 = j0 + 1
        gwait(j0, rows0_v, g0)
        scat(j0, rows0_v, s0)
        gwait(j1, rows1_v, g1)
        scat(j1, rows1_v, s1)
        swait(j0, rows0_v, s0)

        @pl.when(k < ROWS_PER_SUB // 2 - 1)
        def _():
            gather(j0 + 2, rows0_v, g0)

        swait(j1, rows1_v, s1)

        @pl.when(k < ROWS_PER_SUB // 2 - 1)
        def _():
            gather(j1 + 2, rows1_v, g1)

        return carry

    lax.fori_loop(0, ROWS_PER_SUB // 2, body, 0, unroll=False)
    plsc.subcore_barrier()
    pltpu.sync_copy(acc_sh.at[pl.ds(s * NSLICE, NSLICE)],
                    out_hbm.at[c, pl.ds(s * NSLICE, NSLICE)])


_NB = 1000


def _l0_body(acc_ref, degp_ref, b_ref, h_ref, inv_ref):
    deg = degp_ref[0] + degp_ref[1]
    inv = 1.0 / jnp.maximum(deg, 1.0)
    h = jnp.zeros((_NB, D), jnp.float32)
    for r in range(R):
        h = h + acc_ref[r] * inv[r]
    h_ref[...] = jnp.maximum(h + b_ref[...], 0.0)
    inv_ref[...] = inv


def _l0_dense(acc, degp, b1):
    return pl.pallas_call(
        _l0_body,
        grid=(N // _NB,),
        in_specs=[
            pl.BlockSpec((R, _NB, D), lambda i: (0, i, 0)),
            pl.BlockSpec((NC, R, _NB, 1), lambda i: (0, 0, i, 0)),
            pl.BlockSpec((1, D), lambda i: (0, 0)),
        ],
        out_specs=[
            pl.BlockSpec((_NB, D), lambda i: (i, 0)),
            pl.BlockSpec((R, _NB, 1), lambda i: (0, i, 0)),
        ],
        out_shape=[
            jax.ShapeDtypeStruct((N, D), jnp.float32),
            jax.ShapeDtypeStruct((R, N, 1), jnp.float32),
        ],
    )(acc, degp, b1)


def _mm_body(relu, k, acc_ref, inv_ref, w_ref, b_ref, o_ref):
    y = jnp.zeros((_NB, k), jnp.float32)
    for r in range(R):
        x = acc_ref[r] * inv_ref[r]
        y = y + jnp.dot(x, w_ref[r], preferred_element_type=jnp.float32)
    y = y + b_ref[...]
    o_ref[...] = jnp.maximum(y, 0.0) if relu else y


def _mm_dense(acc, inv, w, b, relu):
    k = w.shape[-1]
    return pl.pallas_call(
        functools.partial(_mm_body, relu, k),
        grid=(N // _NB,),
        in_specs=[
            pl.BlockSpec((R, _NB, D), lambda i: (0, i, 0)),
            pl.BlockSpec((R, _NB, 1), lambda i: (0, i, 0)),
            pl.BlockSpec((R, D, k), lambda i: (0, 0, 0)),
            pl.BlockSpec((1, k), lambda i: (0, 0)),
        ],
        out_specs=pl.BlockSpec((_NB, k), lambda i: (i, 0)),
        out_shape=jax.ShapeDtypeStruct((N, k), jnp.float32),
    )(acc, inv, w, b)


@functools.partial(
    pl.kernel,
    out_type=jax.ShapeDtypeStruct((NC, RNP, 16), jnp.float32),
    mesh=_mesh,
    scratch_types=[
        pltpu.VMEM((ROWS_PER_WRK, LW), jnp.int32),
        pltpu.VMEM((ROWS_PER_WRK, LW), jnp.int32),
        pltpu.VMEM((LW, 16), jnp.float32),
        pltpu.VMEM((LW, 16), jnp.float32),
        pltpu.VMEM_SHARED((RNP, 16), jnp.float32),
        pltpu.SemaphoreType.DMA,
        pltpu.SemaphoreType.DMA,
        pltpu.SemaphoreType.DMA,
        pltpu.SemaphoreType.DMA,
    ],
    compiler_params=pltpu.CompilerParams(use_tc_tiling_on_sc=False),
)
def _scatter16_kernel(table_hbm, srci_hbm, dsti_hbm, zeros_hbm, out_hbm,
                      srci_v, dsti_v, rows0_v, rows1_v, acc_sh,
                      g0, g1, s0, s1):
    c = lax.axis_index("c")
    s = lax.axis_index("s")
    row0 = c * (EROWS // NC) + s * ROWS_PER_WRK
    pltpu.sync_copy(zeros_hbm.at[pl.ds(s * NSLICE, NSLICE)],
                    acc_sh.at[pl.ds(s * NSLICE, NSLICE)])
    pltpu.sync_copy(srci_hbm.at[pl.ds(row0, ROWS_PER_WRK)], srci_v)
    pltpu.sync_copy(dsti_hbm.at[pl.ds(row0, ROWS_PER_WRK)], dsti_v)
    plsc.subcore_barrier()

    def gather(j, buf, sem):
        pltpu.async_copy(table_hbm.at[srci_v.at[j]], buf, sem)

    def gwait(j, buf, sem):
        pltpu.make_async_copy(table_hbm.at[srci_v.at[j]], buf, sem).wait()

    gather(0, rows0_v, g0)
    gather(1, rows1_v, g1)

    def body(k, carry):
        j0 = 2 * k
        j1 = j0 + 1
        gwait(j0, rows0_v, g0)
        pltpu.async_copy(rows0_v, acc_sh.at[dsti_v.at[j0]], s0, add=True)
        gwait(j1, rows1_v, g1)
        pltpu.async_copy(rows1_v, acc_sh.at[dsti_v.at[j1]], s1, add=True)
        pltpu.make_async_copy(rows0_v, acc_sh.at[dsti_v.at[j0]], s0).wait()

        @pl.when(k < ROWS_PER_WRK // 2 - 1)
        def _():
            gather(j0 + 2, rows0_v, g0)

        pltpu.make_async_copy(rows1_v, acc_sh.at[dsti_v.at[j1]], s1).wait()

        @pl.when(k < ROWS_PER_WRK // 2 - 1)
        def _():
            gather(j1 + 2, rows1_v, g1)

        return carry

    lax.fori_loop(0, ROWS_PER_WRK // 2, body, 0, unroll=False)
    plsc.subcore_barrier()
    pltpu.sync_copy(acc_sh.at[pl.ds(s * NSLICE, NSLICE)],
                    out_hbm.at[c, pl.ds(s * NSLICE, NSLICE)])


def _y_body(h_ref, w_ref, y_ref):
    for r in range(R):
        y_ref[r] = jnp.dot(h_ref[...], w_ref[r],
                           preferred_element_type=jnp.float32)


def _y_dense(h1, w3):
    return pl.pallas_call(
        _y_body,
        grid=(N // _NB,),
        in_specs=[
            pl.BlockSpec((_NB, D), lambda i: (i, 0)),
            pl.BlockSpec((R, D, 16), lambda i: (0, 0, 0)),
        ],
        out_specs=pl.BlockSpec((R, _NB, 16), lambda i: (0, i, 0)),
        out_shape=jax.ShapeDtypeStruct((R, N, 16), jnp.float32),
    )(h1, w3)


def _out_body(accp_ref, inv_ref, b_ref, o_ref):
    y = jnp.zeros((_NB, 16), jnp.float32)
    for r in range(R):
        y = y + (accp_ref[0, r] + accp_ref[1, r]) * inv_ref[r]
    o_ref[...] = y + b_ref[...]


def _out_dense(accp, inv, b3):
    return pl.pallas_call(
        _out_body,
        grid=(N // _NB,),
        in_specs=[
            pl.BlockSpec((NC, R, _NB, 16), lambda i: (0, 0, i, 0)),
            pl.BlockSpec((R, _NB, 1), lambda i: (0, i, 0)),
            pl.BlockSpec((1, 16), lambda i: (0, 0)),
        ],
        out_specs=pl.BlockSpec((_NB, 16), lambda i: (i, 0)),
        out_shape=jax.ShapeDtypeStruct((N, 16), jnp.float32),
    )(accp, inv, b3)


def _aggregate(x, srci, dsti, zeros32):
    chunks = []
    for p in range(2):
        table = jnp.concatenate(
            [x[:, 64 * p:64 * p + 32], x[:, 64 * p + 32:64 * p + 64]], axis=0)
        chunks.append(_scatter_kernel(table, srci, dsti, zeros32))
    acc = jnp.concatenate(
        [chunks[0][0, :RN], chunks[0][1, :RN],
         chunks[1][0, :RN], chunks[1][1, :RN]], axis=-1)
    return acc.reshape(R, N, D)


def kernel(edge_src, edge_dst, edge_type, embed, h_bias1, W2, b2, W3, b3):
    pad = EPAD - E
    src_p = jnp.concatenate([edge_src, jnp.zeros((pad,), jnp.int32)])
    srci = jnp.stack([src_p, src_p + N]).reshape(NC, EROWS, LW)
    typ_p = jnp.concatenate([edge_type, jnp.zeros((pad,), jnp.int32)])
    srci2 = (typ_p * N + src_p).reshape(EROWS, LW)
    dst_p = jnp.concatenate(
        [edge_type * N + edge_dst, jnp.full((pad,), RN, jnp.int32)])
    dsti = dst_p.reshape(EROWS, LW)

    zeros32 = jnp.zeros((RNP, 32), jnp.float32)
    zeros8 = jnp.zeros((RNP, 8), jnp.float32)
    ones8 = jnp.ones((LW, 8), jnp.float32)

    degp = _deg_kernel(dsti, ones8, zeros8)
    degp = degp[:, :RN, 0].reshape(NC, R, N, 1)

    acc0 = _aggregate(embed, srci, dsti, zeros32)
    h0, inv = _l0_dense(acc0, degp, h_bias1.reshape(1, D))

    acc1 = _aggregate(h0, srci, dsti, zeros32)
    h1 = _mm_dense(acc1, inv, W2, b2.reshape(1, D), relu=True)

    y = _y_dense(h1, W3).reshape(RN, 16)
    zeros16 = jnp.zeros((RNP, 16), jnp.float32)
    accp = _scatter16_kernel(y, srci2, dsti, zeros16)
    accp = accp[:, :RN].reshape(NC, R, N, 16)
    out = _out_dense(accp, inv, b3.reshape(1, 16))
    return out
